# Initial kernel scaffold; baseline (speedup 1.0000x reference)
import jax
import jax.numpy as jnp
from jax.experimental import pallas as pl


def kernel(input_ids, embedding_table):
    raise NotImplementedError("write your pallas kernel here")


# speedup vs baseline: 1.4769x; 1.4769x over previous
import functools

import jax
import jax.numpy as jnp
from jax import lax
from jax.experimental import pallas as pl
from jax.experimental.pallas import tpu as pltpu
from jax.experimental.pallas import tpu_sc as plsc

CHUNK = 128
GROUP = 8
NUM_WORKERS = 32
NUM_CORES = 2


@jax.jit
def _gather_flat(table, ids2d):
    n_chunks = ids2d.shape[0]
    d = table.shape[1]
    n = n_chunks * CHUNK
    chunks_per_w = n_chunks // NUM_WORKERS
    per_w = chunks_per_w * CHUNK
    n_groups = chunks_per_w // GROUP
    rows_per_group = CHUNK * GROUP

    mesh = plsc.VectorSubcoreMesh(core_axis_name="c", subcore_axis_name="s")

    @functools.partial(
        pl.kernel,
        out_type=jax.ShapeDtypeStruct((n, d), jnp.float32),
        mesh=mesh,
        scratch_types=[
            pltpu.VMEM((chunks_per_w, CHUNK), jnp.int32),
            pltpu.VMEM((rows_per_group, d), jnp.float32),
            pltpu.SemaphoreType.DMA,
        ],
        compiler_params=pltpu.CompilerParams(use_tc_tiling_on_sc=False),
    )
    def k(table_hbm, ids_hbm, out_hbm, idx_v, rows_v, sem):
        wid = lax.axis_index("s") * NUM_CORES + lax.axis_index("c")
        base = wid * per_w
        pltpu.sync_copy(ids_hbm.at[pl.ds(wid * chunks_per_w, chunks_per_w)], idx_v)

        def body(g, carry):
            copies = []
            for j in range(GROUP):
                copies.append(
                    pltpu.async_copy(
                        table_hbm.at[idx_v.at[g * GROUP + j]],
                        rows_v.at[pl.ds(j * CHUNK, CHUNK)],
                        sem,
                    )
                )
            for c in copies:
                c.wait()
            pltpu.sync_copy(
                rows_v,
                out_hbm.at[pl.ds(base + g * rows_per_group, rows_per_group)],
            )
            return carry

        lax.fori_loop(0, n_groups, body, 0)

    return k(table, ids2d)


def kernel(input_ids, embedding_table):
    b, h = input_ids.shape
    d = embedding_table.shape[1]
    ids2d = input_ids.reshape(-1, CHUNK).astype(jnp.int32)
    out = _gather_flat(embedding_table, ids2d)
    return out.reshape(b, h, d)


# speedup vs baseline: 1.4988x; 1.0148x over previous
import functools

import jax
import jax.numpy as jnp
from jax import lax
from jax.experimental import pallas as pl
from jax.experimental.pallas import tpu as pltpu
from jax.experimental.pallas import tpu_sc as plsc

CHUNK = 128
GROUP = 10
NUM_WORKERS = 32
NUM_CORES = 2


@jax.jit
def _gather_flat(table, ids2d):
    n_chunks = ids2d.shape[0]
    d = table.shape[1]
    n = n_chunks * CHUNK
    chunks_per_w = n_chunks // NUM_WORKERS
    per_w = chunks_per_w * CHUNK
    n_groups = chunks_per_w // GROUP
    n2 = n_groups // 2
    rows_per_group = CHUNK * GROUP

    mesh = plsc.VectorSubcoreMesh(core_axis_name="c", subcore_axis_name="s")

    @functools.partial(
        pl.kernel,
        out_type=jax.ShapeDtypeStruct((n, d), jnp.float32),
        mesh=mesh,
        scratch_types=[
            pltpu.VMEM((chunks_per_w, CHUNK), jnp.int32),
            pltpu.VMEM((rows_per_group, d), jnp.float32),
            pltpu.VMEM((rows_per_group, d), jnp.float32),
            pltpu.SemaphoreType.DMA,
            pltpu.SemaphoreType.DMA,
            pltpu.SemaphoreType.DMA,
            pltpu.SemaphoreType.DMA,
        ],
        compiler_params=pltpu.CompilerParams(use_tc_tiling_on_sc=False),
    )
    def k(table_hbm, ids_hbm, out_hbm, idx_v, r0, r1, sg0, sg1, ss0, ss1):
        wid = lax.axis_index("s") * NUM_CORES + lax.axis_index("c")
        base = wid * per_w

        def fire_gathers(g, rows, sem):
            copies = []
            for j in range(GROUP):
                copies.append(
                    pltpu.async_copy(
                        table_hbm.at[idx_v.at[g * GROUP + j]],
                        rows.at[pl.ds(j * CHUNK, CHUNK)],
                        sem,
                    )
                )
            return copies

        def wait_gathers(rows, sem):
            for j in range(GROUP):
                pltpu.make_async_copy(
                    table_hbm.at[idx_v.at[0]],
                    rows.at[pl.ds(j * CHUNK, CHUNK)],
                    sem,
                ).wait()

        def out_slice(g):
            return out_hbm.at[pl.ds(base + g * rows_per_group, rows_per_group)]

        def fire_store(rows, g, sem):
            pltpu.async_copy(rows, out_slice(g), sem)

        def wait_store(rows, g, sem):
            pltpu.make_async_copy(rows, out_slice(g), sem).wait()

        pltpu.sync_copy(ids_hbm.at[pl.ds(wid * chunks_per_w, chunks_per_w)], idx_v)

        fire_gathers(0, r0, sg0)

        def body(g2, carry):
            ga = 2 * g2
            gb = ga + 1

            @pl.when(g2 > 0)
            def _():
                wait_store(r1, ga - 1, ss1)

            fire_gathers(gb, r1, sg1)
            wait_gathers(r0, sg0)
            fire_store(r0, ga, ss0)

            wait_store(r0, ga, ss0)

            @pl.when(gb + 1 < n_groups)
            def _():
                fire_gathers(gb + 1, r0, sg0)

            wait_gathers(r1, sg1)
            fire_store(r1, gb, ss1)
            return carry

        lax.fori_loop(0, n2, body, 0)
        wait_store(r1, n_groups - 1, ss1)

    return k(table, ids2d)


def kernel(input_ids, embedding_table):
    b, h = input_ids.shape
    d = embedding_table.shape[1]
    ids2d = input_ids.reshape(-1, CHUNK).astype(jnp.int32)
    out = _gather_flat(embedding_table, ids2d)
    return out.reshape(b, h, d)


# speedup vs baseline: 1.5009x; 1.0014x over previous
import functools

import jax
import jax.numpy as jnp
from jax import lax
from jax.experimental import pallas as pl
from jax.experimental.pallas import tpu as pltpu
from jax.experimental.pallas import tpu_sc as plsc

GROUP_ROWS = 1280
NUM_WORKERS = 32
NUM_CORES = 2


@jax.jit
def _gather_flat(table, ids):
    n = ids.shape[0]
    d = table.shape[1]
    per_w = n // NUM_WORKERS
    n_groups = per_w // GROUP_ROWS
    n2 = n_groups // 2

    mesh = plsc.VectorSubcoreMesh(core_axis_name="c", subcore_axis_name="s")

    @functools.partial(
        pl.kernel,
        out_type=jax.ShapeDtypeStruct((n, d), jnp.float32),
        mesh=mesh,
        scratch_types=[
            pltpu.VMEM((per_w,), jnp.int32),
            pltpu.VMEM((GROUP_ROWS, d), jnp.float32),
            pltpu.VMEM((GROUP_ROWS, d), jnp.float32),
            pltpu.SemaphoreType.DMA,
            pltpu.SemaphoreType.DMA,
            pltpu.SemaphoreType.DMA,
            pltpu.SemaphoreType.DMA,
        ],
        compiler_params=pltpu.CompilerParams(use_tc_tiling_on_sc=False),
    )
    def k(table_hbm, ids_hbm, out_hbm, idx_v, r0, r1, sg0, sg1, ss0, ss1):
        wid = lax.axis_index("s") * NUM_CORES + lax.axis_index("c")
        base = wid * per_w

        def idx_slice(g):
            return idx_v.at[pl.ds(g * GROUP_ROWS, GROUP_ROWS)]

        def fire_gather(g, rows, sem):
            pltpu.async_copy(table_hbm.at[idx_slice(g)], rows, sem)

        def wait_gather(rows, sem):
            pltpu.make_async_copy(table_hbm.at[idx_slice(0)], rows, sem).wait()

        def out_slice(g):
            return out_hbm.at[pl.ds(base + g * GROUP_ROWS, GROUP_ROWS)]

        def fire_store(rows, g, sem):
            pltpu.async_copy(rows, out_slice(g), sem)

        def wait_store(rows, g, sem):
            pltpu.make_async_copy(rows, out_slice(g), sem).wait()

        pltpu.sync_copy(ids_hbm.at[pl.ds(base, per_w)], idx_v)

        fire_gather(0, r0, sg0)

        def body(g2, carry):
            ga = 2 * g2
            gb = ga + 1

            @pl.when(g2 > 0)
            def _():
                wait_store(r1, ga - 1, ss1)

            fire_gather(gb, r1, sg1)
            wait_gather(r0, sg0)
            fire_store(r0, ga, ss0)

            wait_store(r0, ga, ss0)

            @pl.when(gb + 1 < n_groups)
            def _():
                fire_gather(gb + 1, r0, sg0)

            wait_gather(r1, sg1)
            fire_store(r1, gb, ss1)
            return carry

        lax.fori_loop(0, n2, body, 0)
        wait_store(r1, n_groups - 1, ss1)

    return k(table, ids)


def kernel(input_ids, embedding_table):
    b, h = input_ids.shape
    d = embedding_table.shape[1]
    ids = input_ids.reshape(-1).astype(jnp.int32)
    out = _gather_flat(embedding_table, ids)
    return out.reshape(b, h, d)
